# baseline (speedup 1.0000x reference)
import jax
import jax.numpy as jnp
from jax import lax
from jax.experimental import pallas as pl
from jax.experimental.pallas import tpu as pltpu
from jax.experimental.pallas import tpu_sc as plsc

B = 16384
D = 16
L = 50
NE = 8
NP = NE + L
F = (NE + 1) * D
NC = 2
NS = 16
NW = NC * NS
S = B // NW
C = 128
NCH = S // C
LP = 56
NG = 15

HIST_COL = NE * D


CS = 32
NSUB = C // CS
NQ = NP * NSUB
Q_ST = (NE + 1) * NSUB


def _sc_gather_body(idx8_hbm, hist_hbm,
                    emb_user, emb_item, ec0, ec1, ec2, ec3, ec4, ec5,
                    emb_hist, dummy_hbm,
                    feats_hbm,
                    idx8_v, histp_v, slab_v, feats_v,
                    sem0, sem1):
    tables = (emb_user, emb_item, ec0, ec1, ec2, ec3, ec4, ec5)
    sems = (sem0, sem1)
    wid = lax.axis_index("s") * NC + lax.axis_index("c")

    def idx_row(p):
        if isinstance(p, int) and p < NE:
            return idx8_v.at[p]
        return histp_v.at[p - NE]

    def fire(q, src, b):
        p, sub = q // 4, q % 4
        row = idx_row(p)
        rbase = sub * CS

        @pl.loop(0, CS // 16)
        def _fire(rg):
            w = row[pl.ds(rbase + rg * 16, 16)]
            for j in range(16):
                start = pl.multiple_of(w[j] & -8, 8)
                pltpu.async_copy(src.at[pl.ds(start, 8), :],
                                 slab_v.at[b, rg * 16 + j], sems[b])

    def drain(b):
        pltpu.make_async_copy(dummy_hbm, slab_v.at[b], sems[b]).wait()

    def select(q, col, accumulate, b):
        p, sub = q // 4, q % 4
        row = idx_row(p)
        rbase = sub * CS

        @pl.loop(0, CS // 16)
        def _sel(rg):
            w = row[pl.ds(rbase + rg * 16, 16)] & 7
            for j in range(16):
                v = slab_v[b, rg * 16 + j, w[j], :]
                dst = feats_v.at[rbase + rg * 16 + j, pl.ds(col, D)]
                if accumulate:
                    plsc.addupdate(dst, v)
                else:
                    feats_v[rbase + rg * 16 + j, pl.ds(col, D)] = v

    @pl.loop(0, NCH)
    def _chunk(c):
        base = wid * S + c * C

        pltpu.sync_copy(idx8_hbm.at[:, pl.ds(base, C)], idx8_v)
        pltpu.sync_copy(hist_hbm.at[:, pl.ds(base, C)], histp_v)

        def q_src(q):
            p = q // 4
            return tables[p] if p < NE else emb_hist

        fire(0, q_src(0), 0)
        fire(1, q_src(1), 1)
        for q in range(Q_ST):
            p = q // 4
            drain(q % 2)
            if p < NE:
                select(q, p * D, False, q % 2)
            else:
                select(q, HIST_COL, False, q % 2)
            fire(q + 2, q_src(q + 2), q % 2)

        @pl.loop(0, (NQ - Q_ST) // 2)
        def _step(qi):
            q0 = Q_ST + 2 * qi

            for dq in range(2):
                q = q0 + dq
                b = (Q_ST + dq) % 2
                drain(b)
                select(q, HIST_COL, True, b)

                @pl.when(q + 2 < NQ)
                def _f(q=q, b=b):
                    fire(q + 2, emb_hist, b)

        pltpu.sync_copy(feats_v, feats_hbm.at[pl.ds(base, C), :])


def _sc_gather(idx8, histT, emb_user, emb_item, ec0, ec1, ec2, ec3, ec4,
               ec5, emb_hist, dummy):
    mesh = plsc.VectorSubcoreMesh(core_axis_name="c", subcore_axis_name="s")
    return pl.kernel(
        _sc_gather_body,
        out_type=jax.ShapeDtypeStruct((B, F), jnp.float32),
        mesh=mesh,
        scratch_types=[
            pltpu.VMEM((NE, C), jnp.int32),
            pltpu.VMEM((LP, C), jnp.int32),
            pltpu.VMEM((2, CS, 8, D), jnp.float32),
            pltpu.VMEM((C, F), jnp.float32),
            pltpu.SemaphoreType.DMA,
            pltpu.SemaphoreType.DMA,
        ],
        compiler_params=pltpu.CompilerParams(use_tc_tiling_on_sc=True,
                                             needs_layout_passes=False),
    )(idx8, histT, emb_user, emb_item, ec0, ec1, ec2, ec3, ec4, ec5,
      emb_hist, dummy)


def _mlp_body(x_ref, ctn_ref, wv_ref, W1_ref, b1_ref, W2_ref, b2_ref,
              W3_ref, b3_ref, o_ref):
    x = x_ref[...]
    h = jnp.maximum(x @ W1_ref[...] + b1_ref[...][None, :], 0.0)
    h = jnp.maximum(h @ W2_ref[...] + b2_ref[...][None, :], 0.0)
    z = h @ W3_ref[...]
    lin = ctn_ref[...] @ wv_ref[...]
    r = z[:, 0] + lin[:, 0] + b3_ref[0]
    o_ref[...] = jax.nn.sigmoid(r)


def _mlp(feats, ctn, wvec, W1, b1, W2, b2, W3, b3):
    bm = 2048
    grid = (B // bm,)
    return pl.pallas_call(
        _mlp_body,
        grid=grid,
        in_specs=[
            pl.BlockSpec((bm, F), lambda i: (i, 0)),
            pl.BlockSpec((bm, 4), lambda i: (i, 0)),
            pl.BlockSpec((4, 1), lambda i: (0, 0)),
            pl.BlockSpec((F, 256), lambda i: (0, 0)),
            pl.BlockSpec((256,), lambda i: (0,)),
            pl.BlockSpec((256, 128), lambda i: (0, 0)),
            pl.BlockSpec((128,), lambda i: (0,)),
            pl.BlockSpec((128, 1), lambda i: (0, 0)),
            pl.BlockSpec((1,), lambda i: (0,)),
        ],
        out_specs=pl.BlockSpec((bm,), lambda i: (i,)),
        out_shape=jax.ShapeDtypeStruct((B,), jnp.float32),
    )(feats, ctn, wvec, W1, b1, W2, b2, W3, b3)


def kernel(user_id, item_id, cat_0, cat_1, cat_2, cat_3, cat_4, cat_5,
           ctn_0, ctn_1, ctn_2, ctn_3, hist_item,
           emb_user, emb_item, emb_cat_0, emb_cat_1, emb_cat_2, emb_cat_3,
           emb_cat_4, emb_cat_5, emb_hist,
           w_ctn_0, w_ctn_1, w_ctn_2, w_ctn_3,
           W1, b1, W2, b2, W3, b3):
    idx8 = jnp.stack([
        user_id[:, 0], item_id[:, 0], cat_0[:, 0], cat_1[:, 0],
        cat_2[:, 0], cat_3[:, 0], cat_4[:, 0], cat_5[:, 0],
    ]).astype(jnp.int32)
    histT = hist_item.T.astype(jnp.int32)
    histT = jnp.pad(histT, ((0, LP - L), (0, 0)))
    dummy = jnp.zeros((CS, 8, D), jnp.float32)

    feats = _sc_gather(idx8, histT, emb_user, emb_item, emb_cat_0,
                       emb_cat_1, emb_cat_2, emb_cat_3, emb_cat_4,
                       emb_cat_5, emb_hist, dummy)

    ctn = jnp.concatenate([ctn_0, ctn_1, ctn_2, ctn_3], axis=1)
    wvec = jnp.stack([w_ctn_0[0, 0], w_ctn_1[0, 0], w_ctn_2[0, 0],
                      w_ctn_3[0, 0]]).reshape(4, 1)
    return _mlp(feats, ctn, wvec, W1, b1, W2, b2, W3, b3)
